# initial kernel scaffold (unmeasured)
import jax
import jax.numpy as jnp
from jax import lax
from jax.experimental import pallas as pl
from jax.experimental.pallas import tpu as pltpu

B = 8
SQ = 8
H = 16
D = 128
SKV_LOCAL = 1024
SCALE = D ** -0.5


def kernel(Q, K, V):
    def body(q_ref, k_ref, v_ref, out_ref,
             o_acc, m_acc, l_acc, o_rcv, m_rcv, l_rcv,
             send_sems, recv_sems):
        b = pl.program_id(0)
        h = pl.program_id(1)
        i = b * H + h

        qb = q_ref[0, :, 0, :].astype(jnp.bfloat16)
        kb = k_ref[0, :, 0, :].astype(jnp.bfloat16)
        vb = v_ref[0, :, 0, :].astype(jnp.bfloat16)

        s = lax.dot_general(
            qb, kb, (((1,), (1,)), ((), ())),
            preferred_element_type=jnp.float32,
        ) * SCALE
        m = jnp.max(s, axis=1)
        p = jnp.exp(s - m[:, None])
        l = jnp.sum(p, axis=1)
        o = lax.dot_general(
            p.astype(jnp.bfloat16), vb, (((1,), (0,)), ((), ())),
            preferred_element_type=jnp.float32,
        )

        o_acc[i, :, :] = o
        m_acc[pl.ds(i, 1), :] = m.reshape(1, SQ)
        l_acc[pl.ds(i, 1), :] = l.reshape(1, SQ)

        @pl.when(jnp.logical_and(b == B - 1, h == H - 1))
        def _():
            my_x = lax.axis_index("x")
            my_y = lax.axis_index("y")
            partner = (my_x, 1 - my_y)

            copies = [
                pltpu.make_async_remote_copy(
                    src_ref=src, dst_ref=dst,
                    send_sem=send_sems.at[j], recv_sem=recv_sems.at[j],
                    device_id=partner, device_id_type=pl.DeviceIdType.MESH,
                )
                for j, (src, dst) in enumerate(
                    [(o_acc, o_rcv), (m_acc, m_rcv), (l_acc, l_rcv)]
                )
            ]
            for c in copies:
                c.start()
            for c in copies:
                c.wait()

            m_a = m_acc[...]
            m_b = m_rcv[...]
            m_new = jnp.maximum(m_a, m_b)
            wa = jnp.exp(m_a - m_new)
            wb = jnp.exp(m_b - m_new)
            l_new = wa * l_acc[...] + wb * l_rcv[...]
            o_new = (wa[:, :, None] * o_acc[...]
                     + wb[:, :, None] * o_rcv[...]) / l_new[:, :, None]
            out = o_new.reshape(B, H, SQ, D).transpose(0, 2, 1, 3)
            out_ref[...] = out.astype(jnp.float32)

    return pl.pallas_call(
        body,
        grid=(B, H),
        in_specs=[
            pl.BlockSpec((1, SQ, 1, D), lambda b, h: (b, 0, h, 0)),
            pl.BlockSpec((1, SKV_LOCAL, 1, D), lambda b, h: (b, 0, h, 0)),
            pl.BlockSpec((1, SKV_LOCAL, 1, D), lambda b, h: (b, 0, h, 0)),
        ],
        out_specs=pl.BlockSpec((B, SQ, H, D), lambda b, h: (0, 0, 0, 0)),
        out_shape=jax.ShapeDtypeStruct((B, SQ, H, D), jnp.float32),
        scratch_shapes=[
            pltpu.VMEM((B * H, SQ, D), jnp.float32),
            pltpu.VMEM((B * H, SQ), jnp.float32),
            pltpu.VMEM((B * H, SQ), jnp.float32),
            pltpu.VMEM((B * H, SQ, D), jnp.float32),
            pltpu.VMEM((B * H, SQ), jnp.float32),
            pltpu.VMEM((B * H, SQ), jnp.float32),
            pltpu.SemaphoreType.DMA((3,)),
            pltpu.SemaphoreType.DMA((3,)),
        ],
        compiler_params=pltpu.CompilerParams(collective_id=0),
    )(Q, K, V)


# baseline (device time: 162215 ns/iter reference)
import jax
import jax.numpy as jnp
from jax import lax
from jax.experimental import pallas as pl
from jax.experimental.pallas import tpu as pltpu

B = 8
SQ = 8
H = 16
HB = 8
D = 128
SKV_LOCAL = 1024
SCALE = D ** -0.5


def kernel(Q, K, V):
    def body(q_ref, k_ref, v_ref, out_ref,
             o_acc, m_acc, l_acc, o_rcv, m_rcv, l_rcv,
             send_sems, recv_sems):
        b = pl.program_id(0)
        hb = pl.program_id(1)

        h0 = hb * HB
        for h in range(HB):
            qh = q_ref[0, :, h, :].astype(jnp.bfloat16)
            kh = k_ref[0, :, h, :].astype(jnp.bfloat16)
            vh = v_ref[0, :, h, :].astype(jnp.bfloat16)
            s = lax.dot_general(
                qh, kh, (((1,), (1,)), ((), ())),
                preferred_element_type=jnp.float32,
            ) * SCALE
            m = jnp.max(s, axis=1)
            p = jnp.exp(s - m[:, None])
            l = jnp.sum(p, axis=1)
            o = lax.dot_general(
                p.astype(jnp.bfloat16), vh, (((1,), (0,)), ((), ())),
                preferred_element_type=jnp.float32,
            )
            o_acc[b, h0 + h, :, :] = o
            m_acc[b, pl.ds(h0 + h, 1), :] = m.reshape(1, SQ)
            l_acc[b, pl.ds(h0 + h, 1), :] = l.reshape(1, SQ)

        @pl.when(jnp.logical_and(b == B - 1, hb == H // HB - 1))
        def _():
            my_x = lax.axis_index("x")
            my_y = lax.axis_index("y")
            partner = (my_x, 1 - my_y)

            copies = [
                pltpu.make_async_remote_copy(
                    src_ref=src, dst_ref=dst,
                    send_sem=send_sems.at[j], recv_sem=recv_sems.at[j],
                    device_id=partner, device_id_type=pl.DeviceIdType.MESH,
                )
                for j, (src, dst) in enumerate(
                    [(o_acc, o_rcv), (m_acc, m_rcv), (l_acc, l_rcv)]
                )
            ]
            for c in copies:
                c.start()
            for c in copies:
                c.wait()

            m_a = m_acc[...]
            m_b = m_rcv[...]
            m_new = jnp.maximum(m_a, m_b)
            wa = jnp.exp(m_a - m_new)
            wb = jnp.exp(m_b - m_new)
            l_new = wa * l_acc[...] + wb * l_rcv[...]
            o_new = (wa[..., None] * o_acc[...]
                     + wb[..., None] * o_rcv[...]) / l_new[..., None]
            out = o_new.transpose(0, 2, 1, 3)
            out_ref[...] = out.astype(jnp.float32)

    return pl.pallas_call(
        body,
        grid=(B, H // HB),
        in_specs=[
            pl.BlockSpec((1, SQ, HB, D), lambda b, hb: (b, 0, hb, 0)),
            pl.BlockSpec((1, SKV_LOCAL, HB, D), lambda b, hb: (b, 0, hb, 0)),
            pl.BlockSpec((1, SKV_LOCAL, HB, D), lambda b, hb: (b, 0, hb, 0)),
        ],
        out_specs=pl.BlockSpec((B, SQ, H, D), lambda b, hb: (0, 0, 0, 0)),
        out_shape=jax.ShapeDtypeStruct((B, SQ, H, D), jnp.float32),
        scratch_shapes=[
            pltpu.VMEM((B, H, SQ, D), jnp.float32),
            pltpu.VMEM((B, H, SQ), jnp.float32),
            pltpu.VMEM((B, H, SQ), jnp.float32),
            pltpu.VMEM((B, H, SQ, D), jnp.float32),
            pltpu.VMEM((B, H, SQ), jnp.float32),
            pltpu.VMEM((B, H, SQ), jnp.float32),
            pltpu.SemaphoreType.DMA((3,)),
            pltpu.SemaphoreType.DMA((3,)),
        ],
    )(Q, K, V)


# device time: 104321 ns/iter; 1.5550x vs baseline; 1.5550x over previous
import jax
import jax.numpy as jnp
from jax import lax
from jax.experimental import pallas as pl
from jax.experimental.pallas import tpu as pltpu

B = 8
SQ = 8
H = 16
HB = 8
D = 128
SKV_LOCAL = 1024
SKV_HALF = SKV_LOCAL // 2
SCALE = D ** -0.5


def kernel(Q, K, V):
    def body(q_ref, k_ref, v_ref, out_ref,
             o_acc, m_acc, l_acc,
             o_rcv1, m_rcv1, l_rcv1,
             o_rcv2, m_rcv2, l_rcv2,
             send_sems, recv_sems):
        b = pl.program_id(0)
        hb = pl.program_id(1)

        h0 = hb * HB
        for h in range(HB):
            qh = q_ref[0, :, h, :].astype(jnp.bfloat16)
            kh = k_ref[0, :, h, :].astype(jnp.bfloat16)
            vh = v_ref[0, :, h, :].astype(jnp.bfloat16)
            s = lax.dot_general(
                qh, kh, (((1,), (1,)), ((), ())),
                preferred_element_type=jnp.float32,
            ) * SCALE
            m = jnp.max(s, axis=1)
            p = jnp.exp(s - m[:, None])
            l = jnp.sum(p, axis=1)
            o = lax.dot_general(
                p.astype(jnp.bfloat16), vh, (((1,), (0,)), ((), ())),
                preferred_element_type=jnp.float32,
            )
            o_acc[b, h0 + h, :, :] = o
            m_acc[b, pl.ds(h0 + h, 1), :] = m.reshape(1, SQ)
            l_acc[b, pl.ds(h0 + h, 1), :] = l.reshape(1, SQ)

        @pl.when(jnp.logical_and(b == B - 1, hb == H // HB - 1))
        def _():
            my_x = lax.axis_index("x")
            my_y = lax.axis_index("y")

            def exchange(partner, dsts, sem0):
                copies = [
                    pltpu.make_async_remote_copy(
                        src_ref=src, dst_ref=dst,
                        send_sem=send_sems.at[sem0 + j],
                        recv_sem=recv_sems.at[sem0 + j],
                        device_id=partner,
                        device_id_type=pl.DeviceIdType.MESH,
                    )
                    for j, (src, dst) in enumerate(
                        zip([o_acc, m_acc, l_acc], dsts)
                    )
                ]
                for c in copies:
                    c.start()
                for c in copies:
                    c.wait()

            exchange((1 - my_x, my_y), [o_rcv1, m_rcv1, l_rcv1], 0)
            m1 = jnp.maximum(m_acc[...], m_rcv1[...])
            wa = jnp.exp(m_acc[...] - m1)
            wb = jnp.exp(m_rcv1[...] - m1)
            l1 = wa * l_acc[...] + wb * l_rcv1[...]
            o1 = wa[..., None] * o_acc[...] + wb[..., None] * o_rcv1[...]
            m_acc[...] = m1
            l_acc[...] = l1
            o_acc[...] = o1

            exchange((my_x, 1 - my_y), [o_rcv2, m_rcv2, l_rcv2], 3)
            m2 = jnp.maximum(m_acc[...], m_rcv2[...])
            wc = jnp.exp(m_acc[...] - m2)
            wd = jnp.exp(m_rcv2[...] - m2)
            l2 = wc * l_acc[...] + wd * l_rcv2[...]
            o2 = (wc[..., None] * o_acc[...]
                  + wd[..., None] * o_rcv2[...]) / l2[..., None]
            out = o2.transpose(0, 2, 1, 3)
            out_ref[...] = out.astype(jnp.float32)

    def kv_map(bi, hbi):
        return (bi, lax.axis_index("x"), hbi, 0)

    return pl.pallas_call(
        body,
        grid=(B, H // HB),
        in_specs=[
            pl.BlockSpec((1, SQ, HB, D), lambda bi, hbi: (bi, 0, hbi, 0)),
            pl.BlockSpec((1, SKV_HALF, HB, D), kv_map),
            pl.BlockSpec((1, SKV_HALF, HB, D), kv_map),
        ],
        out_specs=pl.BlockSpec((B, SQ, H, D), lambda bi, hbi: (0, 0, 0, 0)),
        out_shape=jax.ShapeDtypeStruct((B, SQ, H, D), jnp.float32),
        scratch_shapes=[
            pltpu.VMEM((B, H, SQ, D), jnp.float32),
            pltpu.VMEM((B, H, SQ), jnp.float32),
            pltpu.VMEM((B, H, SQ), jnp.float32),
            pltpu.VMEM((B, H, SQ, D), jnp.float32),
            pltpu.VMEM((B, H, SQ), jnp.float32),
            pltpu.VMEM((B, H, SQ), jnp.float32),
            pltpu.VMEM((B, H, SQ, D), jnp.float32),
            pltpu.VMEM((B, H, SQ), jnp.float32),
            pltpu.VMEM((B, H, SQ), jnp.float32),
            pltpu.SemaphoreType.DMA((6,)),
            pltpu.SemaphoreType.DMA((6,)),
        ],
    )(Q, K, V)
